# baseline (speedup 1.0000x reference)
import jax
import jax.numpy as jnp
from jax.experimental import pallas as pl
from jax.experimental.pallas import tpu as pltpu

_NCLS = 1000
_MAXP = 100
_FDIM = 128
_CB = 8


def _body(cid_ref, feat_ref, protos_ref, counts_ref, protos_out, counts_out):
    i = pl.program_id(0)
    protos_out[...] = protos_ref[...]
    counts_out[...] = counts_ref[...]
    cid = cid_ref[0]
    base = i * _CB

    @pl.when((cid >= base) & (cid < base + _CB))
    def _():
        f = feat_ref[...]
        norm = jnp.sqrt(jnp.sum(f * f, axis=1, keepdims=True))
        fn = f / jnp.maximum(norm, 1e-12)
        local = cid - base
        protos_out[pl.ds(local, 1)] = fn[:_MAXP][None]
        counts_out[pl.ds(local, 1)] = jnp.ones((1, _MAXP), jnp.int32)


def kernel(features, prototypes, counts, class_id):
    cid = jnp.atleast_1d(jnp.asarray(class_id, jnp.int32))
    grid_spec = pltpu.PrefetchScalarGridSpec(
        num_scalar_prefetch=1,
        grid=(_NCLS // _CB,),
        in_specs=[
            pl.BlockSpec((104, _FDIM), lambda i, s: (0, 0)),
            pl.BlockSpec((_CB, _MAXP, _FDIM), lambda i, s: (i, 0, 0)),
            pl.BlockSpec((_CB, _MAXP), lambda i, s: (i, 0)),
        ],
        out_specs=[
            pl.BlockSpec((_CB, _MAXP, _FDIM), lambda i, s: (i, 0, 0)),
            pl.BlockSpec((_CB, _MAXP), lambda i, s: (i, 0)),
        ],
    )
    return pl.pallas_call(
        _body,
        grid_spec=grid_spec,
        out_shape=(
            jax.ShapeDtypeStruct((_NCLS, _MAXP, _FDIM), jnp.float32),
            jax.ShapeDtypeStruct((_NCLS, _MAXP), jnp.int32),
        ),
        compiler_params=pltpu.CompilerParams(
            dimension_semantics=("arbitrary",),
        ),
    )(cid, features, prototypes, counts)
